# initial kernel scaffold (unmeasured)
import jax
import jax.numpy as jnp
from jax import lax
from jax.experimental import pallas as pl
from jax.experimental.pallas import tpu as pltpu


def kernel(
    x,
):
    def body(*refs):
        pass

    out_shape = jax.ShapeDtypeStruct(..., jnp.float32)
    return pl.pallas_call(body, out_shape=out_shape)(...)


# baseline (device time: 1169796 ns/iter reference)
import jax
import jax.numpy as jnp
from jax import lax
from jax.experimental import pallas as pl
from jax.experimental.pallas import tpu as pltpu

N_DEV = 4


def kernel(x):
    m_per, n = x.shape
    ch = m_per // N_DEV

    def body(x_hbm, out_hbm, comm, x_stage, send_sems, recv_sems, copy_sem,
             credit_sem):
        my = lax.axis_index("i")
        left = lax.rem(my + N_DEV - 1, N_DEV)
        right = lax.rem(my + 1, N_DEV)

        barrier = pltpu.get_barrier_semaphore()
        for nbr in (left, right):
            pl.semaphore_signal(barrier, inc=1, device_id=(nbr,),
                                device_id_type=pl.DeviceIdType.MESH)
        pl.semaphore_wait(barrier, 2)

        pl.semaphore_signal(credit_sem, inc=1, device_id=(left,),
                            device_id_type=pl.DeviceIdType.MESH)

        cp = pltpu.make_async_copy(
            x_hbm.at[pl.ds(my * ch, ch), :], comm.at[0], copy_sem)
        cp.start()
        cp.wait()

        for s in range(2 * (N_DEV - 1)):
            send_slot = s % 2
            recv_slot = (s + 1) % 2
            rdma = pltpu.make_async_remote_copy(
                src_ref=comm.at[send_slot],
                dst_ref=comm.at[recv_slot],
                send_sem=send_sems.at[send_slot],
                recv_sem=recv_sems.at[recv_slot],
                device_id=(right,),
                device_id_type=pl.DeviceIdType.MESH,
            )
            pl.semaphore_wait(credit_sem, 1)
            rdma.start()

            if s < N_DEV - 1:
                idx = lax.rem(my - s - 1 + 2 * N_DEV, N_DEV)
                cpx = pltpu.make_async_copy(
                    x_hbm.at[pl.ds(idx * ch, ch), :], x_stage, copy_sem)
                cpx.start()

            rdma.wait_send()
            if s < 2 * (N_DEV - 1) - 1:
                pl.semaphore_signal(credit_sem, inc=1, device_id=(left,),
                                    device_id_type=pl.DeviceIdType.MESH)
            rdma.wait_recv()

            if s < N_DEV - 1:
                cpx.wait()
                comm[recv_slot] = comm[recv_slot] + x_stage[...]
                if s == N_DEV - 2:
                    idx_out = lax.rem(my + 1, N_DEV)
                    cpo = pltpu.make_async_copy(
                        comm.at[recv_slot],
                        out_hbm.at[pl.ds(idx_out * ch, ch), :], copy_sem)
                    cpo.start()
                    cpo.wait()
            else:
                idx_out = lax.rem(my - (s - (N_DEV - 1)) + 2 * N_DEV, N_DEV)
                cpo = pltpu.make_async_copy(
                    comm.at[recv_slot],
                    out_hbm.at[pl.ds(idx_out * ch, ch), :], copy_sem)
                cpo.start()
                cpo.wait()

    return pl.pallas_call(
        body,
        out_shape=jax.ShapeDtypeStruct((m_per, n), x.dtype),
        in_specs=[pl.BlockSpec(memory_space=pltpu.MemorySpace.HBM)],
        out_specs=pl.BlockSpec(memory_space=pltpu.MemorySpace.HBM),
        scratch_shapes=[
            pltpu.VMEM((2, ch, n), x.dtype),
            pltpu.VMEM((ch, n), x.dtype),
            pltpu.SemaphoreType.DMA((2,)),
            pltpu.SemaphoreType.DMA((2,)),
            pltpu.SemaphoreType.DMA,
            pltpu.SemaphoreType.REGULAR,
        ],
        compiler_params=pltpu.CompilerParams(
            collective_id=0,
            vmem_limit_bytes=100 * 1024 * 1024,
        ),
    )(x)


# device time: 613059 ns/iter; 1.9081x vs baseline; 1.9081x over previous
import jax
import jax.numpy as jnp
from jax import lax
from jax.experimental import pallas as pl
from jax.experimental.pallas import tpu as pltpu

N_DEV = 4


def kernel(x):
    m_per, n = x.shape
    ch = m_per // N_DEV
    nh = n // 2

    def body(x_hbm, out_hbm, comm_r, comm_l, stage_r, stage_l,
             send_sems_r, recv_sems_r, send_sems_l, recv_sems_l,
             copy_sems, cred_r, cred_l):
        my = lax.axis_index("i")
        left = lax.rem(my + N_DEV - 1, N_DEV)
        right = lax.rem(my + 1, N_DEV)

        barrier = pltpu.get_barrier_semaphore()
        for nbr in (left, right):
            pl.semaphore_signal(barrier, inc=1, device_id=(nbr,),
                                device_id_type=pl.DeviceIdType.MESH)
        pl.semaphore_wait(barrier, 2)

        pl.semaphore_signal(cred_r, inc=1, device_id=(left,),
                            device_id_type=pl.DeviceIdType.MESH)
        pl.semaphore_signal(cred_l, inc=1, device_id=(right,),
                            device_id_type=pl.DeviceIdType.MESH)

        seed_r = pltpu.make_async_copy(
            x_hbm.at[pl.ds(my * ch, ch), pl.ds(0, nh)],
            comm_r.at[0], copy_sems.at[0])
        seed_l = pltpu.make_async_copy(
            x_hbm.at[pl.ds(my * ch, ch), pl.ds(nh, nh)],
            comm_l.at[0], copy_sems.at[1])
        seed_r.start()
        seed_l.start()
        seed_r.wait()
        seed_l.wait()

        pend_store_r = [None]
        pend_store_l = [None]

        for s in range(2 * (N_DEV - 1)):
            send_slot = s % 2
            recv_slot = (s + 1) % 2

            pl.semaphore_wait(cred_r, 1)
            rdma_r = pltpu.make_async_remote_copy(
                src_ref=comm_r.at[send_slot],
                dst_ref=comm_r.at[recv_slot],
                send_sem=send_sems_r.at[send_slot],
                recv_sem=recv_sems_r.at[recv_slot],
                device_id=(right,),
                device_id_type=pl.DeviceIdType.MESH,
            )
            rdma_r.start()

            pl.semaphore_wait(cred_l, 1)
            rdma_l = pltpu.make_async_remote_copy(
                src_ref=comm_l.at[send_slot],
                dst_ref=comm_l.at[recv_slot],
                send_sem=send_sems_l.at[send_slot],
                recv_sem=recv_sems_l.at[recv_slot],
                device_id=(left,),
                device_id_type=pl.DeviceIdType.MESH,
            )
            rdma_l.start()

            if s < N_DEV - 1:
                idx_r = lax.rem(my - s - 1 + 2 * N_DEV, N_DEV)
                idx_l = lax.rem(my + s + 1, N_DEV)
                cpx_r = pltpu.make_async_copy(
                    x_hbm.at[pl.ds(idx_r * ch, ch), pl.ds(0, nh)],
                    stage_r, copy_sems.at[0])
                cpx_l = pltpu.make_async_copy(
                    x_hbm.at[pl.ds(idx_l * ch, ch), pl.ds(nh, nh)],
                    stage_l, copy_sems.at[1])
                cpx_r.start()
                cpx_l.start()

            rdma_r.wait_send()
            if pend_store_r[0] is not None:
                pend_store_r[0].wait()
                pend_store_r[0] = None
            rdma_l.wait_send()
            if pend_store_l[0] is not None:
                pend_store_l[0].wait()
                pend_store_l[0] = None
            if s < 2 * (N_DEV - 1) - 1:
                pl.semaphore_signal(cred_r, inc=1, device_id=(left,),
                                    device_id_type=pl.DeviceIdType.MESH)
                pl.semaphore_signal(cred_l, inc=1, device_id=(right,),
                                    device_id_type=pl.DeviceIdType.MESH)

            rdma_r.wait_recv()
            rdma_l.wait_recv()

            if s < N_DEV - 1:
                cpx_r.wait()
                comm_r[recv_slot] = comm_r[recv_slot] + stage_r[...]
                cpx_l.wait()
                comm_l[recv_slot] = comm_l[recv_slot] + stage_l[...]
                if s == N_DEV - 2:
                    out_r = lax.rem(my + 1, N_DEV)
                    out_l = lax.rem(my + N_DEV - 1, N_DEV)
                else:
                    continue
            else:
                t = s - (N_DEV - 1)
                out_r = lax.rem(my - t + 2 * N_DEV, N_DEV)
                out_l = lax.rem(my + t, N_DEV)

            cpo_r = pltpu.make_async_copy(
                comm_r.at[recv_slot],
                out_hbm.at[pl.ds(out_r * ch, ch), pl.ds(0, nh)],
                copy_sems.at[2])
            cpo_l = pltpu.make_async_copy(
                comm_l.at[recv_slot],
                out_hbm.at[pl.ds(out_l * ch, ch), pl.ds(nh, nh)],
                copy_sems.at[3])
            cpo_r.start()
            cpo_l.start()
            pend_store_r[0] = cpo_r
            pend_store_l[0] = cpo_l

        pend_store_r[0].wait()
        pend_store_l[0].wait()

    return pl.pallas_call(
        body,
        out_shape=jax.ShapeDtypeStruct((m_per, n), x.dtype),
        in_specs=[pl.BlockSpec(memory_space=pltpu.MemorySpace.HBM)],
        out_specs=pl.BlockSpec(memory_space=pltpu.MemorySpace.HBM),
        scratch_shapes=[
            pltpu.VMEM((2, ch, nh), x.dtype),
            pltpu.VMEM((2, ch, nh), x.dtype),
            pltpu.VMEM((ch, nh), x.dtype),
            pltpu.VMEM((ch, nh), x.dtype),
            pltpu.SemaphoreType.DMA((2,)),
            pltpu.SemaphoreType.DMA((2,)),
            pltpu.SemaphoreType.DMA((2,)),
            pltpu.SemaphoreType.DMA((2,)),
            pltpu.SemaphoreType.DMA((4,)),
            pltpu.SemaphoreType.REGULAR,
            pltpu.SemaphoreType.REGULAR,
        ],
        compiler_params=pltpu.CompilerParams(
            collective_id=0,
            vmem_limit_bytes=100 * 1024 * 1024,
        ),
    )(x)


# device time: 599857 ns/iter; 1.9501x vs baseline; 1.0220x over previous
import jax
import jax.numpy as jnp
from jax import lax
from jax.experimental import pallas as pl
from jax.experimental.pallas import tpu as pltpu

N_DEV = 4
K = 4
N_HOP = 2 * (N_DEV - 1)


def kernel(x):
    m_per, n = x.shape
    ch = m_per // N_DEV
    ck = ch // K
    nh = n // 2

    def body(x_hbm, out_hbm, comm_r, comm_l, stage_r, stage_l,
             ss_r, rs_r, ld_r, st_r, ss_l, rs_l, ld_l, st_l,
             cred_r, cred_l):
        my = lax.axis_index("i")
        left = lax.rem(my + N_DEV - 1, N_DEV)
        right = lax.rem(my + 1, N_DEV)

        dirs = [
            dict(d=1, comm=comm_r, stage=stage_r, ss=ss_r, rs=rs_r,
                 ld=ld_r, st=st_r, cred=cred_r, to=right, fr=left,
                 col=0),
            dict(d=-1, comm=comm_l, stage=stage_l, ss=ss_l, rs=rs_l,
                 ld=ld_l, st=st_l, cred=cred_l, to=left, fr=right,
                 col=nh),
        ]

        def add_idx(c, s):
            return lax.rem(my - c["d"] * (s + 1) + 2 * N_DEV, N_DEV)

        def out_idx(c, s):
            if s == N_DEV - 2:
                return lax.rem(my + c["d"] + N_DEV, N_DEV)
            t = s - (N_DEV - 1)
            return lax.rem(my - c["d"] * t + 2 * N_DEV, N_DEV)

        def mk_rdma(c, s, k):
            return pltpu.make_async_remote_copy(
                src_ref=c["comm"].at[s % 2, k],
                dst_ref=c["comm"].at[(s + 1) % 2, k],
                send_sem=c["ss"].at[k],
                recv_sem=c["rs"].at[k],
                device_id=(c["to"],),
                device_id_type=pl.DeviceIdType.MESH,
            )

        def mk_load(c, s, k):
            row0 = add_idx(c, s) * ch + k * ck
            return pltpu.make_async_copy(
                x_hbm.at[pl.ds(row0, ck), pl.ds(c["col"], nh)],
                c["stage"].at[k], c["ld"].at[k])

        for c in dirs:
            c["seed"] = []
            for k in range(K):
                cp = pltpu.make_async_copy(
                    x_hbm.at[pl.ds(my * ch + k * ck, ck),
                             pl.ds(c["col"], nh)],
                    c["comm"].at[0, k], c["st"].at[k])
                cp.start()
                c["seed"].append(cp)
            c["pend_ld"] = [mk_load(c, 0, k) for k in range(K)]
            for cp in c["pend_ld"]:
                cp.start()
            c["pend_st"] = [None] * K
            c["rd"] = {}

        barrier = pltpu.get_barrier_semaphore()
        for nbr in (left, right):
            pl.semaphore_signal(barrier, inc=1, device_id=(nbr,),
                                device_id_type=pl.DeviceIdType.MESH)
        pl.semaphore_wait(barrier, 2)

        for c in dirs:
            pl.semaphore_signal(c["cred"], inc=K, device_id=(c["fr"],),
                                device_id_type=pl.DeviceIdType.MESH)

        for c in dirs:
            for cp in c["seed"]:
                cp.wait()
        for c in dirs:
            for k in range(K):
                pl.semaphore_wait(c["cred"], 1)
                c["rd"][(0, k)] = mk_rdma(c, 0, k)
                c["rd"][(0, k)].start()

        for s in range(N_HOP):
            recv_slot = (s + 1) % 2
            for k in range(K):
                for c in dirs:
                    rd = c["rd"].pop((s, k))
                    rd.wait_send()
                    if c["pend_st"][k] is not None:
                        c["pend_st"][k].wait()
                        c["pend_st"][k] = None
                    rd.wait_recv()
                    if s < N_HOP - 1:
                        pl.semaphore_signal(
                            c["cred"], inc=1, device_id=(c["fr"],),
                            device_id_type=pl.DeviceIdType.MESH)
                    if s < N_DEV - 1:
                        c["pend_ld"][k].wait()
                        c["comm"][recv_slot, k] = (
                            c["comm"][recv_slot, k] + c["stage"][k])
                    if s < N_HOP - 1:
                        pl.semaphore_wait(c["cred"], 1)
                        c["rd"][(s + 1, k)] = mk_rdma(c, s + 1, k)
                        c["rd"][(s + 1, k)].start()
                    if s < N_DEV - 2:
                        c["pend_ld"][k] = mk_load(c, s + 1, k)
                        c["pend_ld"][k].start()
                    if s >= N_DEV - 2:
                        row0 = out_idx(c, s) * ch + k * ck
                        cpo = pltpu.make_async_copy(
                            c["comm"].at[recv_slot, k],
                            out_hbm.at[pl.ds(row0, ck),
                                       pl.ds(c["col"], nh)],
                            c["st"].at[k])
                        cpo.start()
                        c["pend_st"][k] = cpo

        for c in dirs:
            for k in range(K):
                c["pend_st"][k].wait()

    return pl.pallas_call(
        body,
        out_shape=jax.ShapeDtypeStruct((m_per, n), x.dtype),
        in_specs=[pl.BlockSpec(memory_space=pltpu.MemorySpace.HBM)],
        out_specs=pl.BlockSpec(memory_space=pltpu.MemorySpace.HBM),
        scratch_shapes=[
            pltpu.VMEM((2, K, ck, nh), x.dtype),
            pltpu.VMEM((2, K, ck, nh), x.dtype),
            pltpu.VMEM((K, ck, nh), x.dtype),
            pltpu.VMEM((K, ck, nh), x.dtype),
            pltpu.SemaphoreType.DMA((K,)),
            pltpu.SemaphoreType.DMA((K,)),
            pltpu.SemaphoreType.DMA((K,)),
            pltpu.SemaphoreType.DMA((K,)),
            pltpu.SemaphoreType.DMA((K,)),
            pltpu.SemaphoreType.DMA((K,)),
            pltpu.SemaphoreType.DMA((K,)),
            pltpu.SemaphoreType.DMA((K,)),
            pltpu.SemaphoreType.REGULAR,
            pltpu.SemaphoreType.REGULAR,
        ],
        compiler_params=pltpu.CompilerParams(
            collective_id=0,
            vmem_limit_bytes=100 * 1024 * 1024,
        ),
    )(x)


# device time: 589782 ns/iter; 1.9834x vs baseline; 1.0171x over previous
import jax
import jax.numpy as jnp
from jax import lax
from jax.experimental import pallas as pl
from jax.experimental.pallas import tpu as pltpu

N_DEV = 4
K = 4
N_HOP = 2 * (N_DEV - 1)


def kernel(x):
    m_per, n = x.shape
    ch = m_per // N_DEV
    ck = ch // K
    nh = n // 2

    def body(x_hbm, out_hbm, comm_r, comm_l, stage_r, stage_l,
             ss_r, rs_r, ld_r, st_r, ss_l, rs_l, ld_l, st_l,
             cred_r, cred_l):
        my = lax.axis_index("i")
        left = lax.rem(my + N_DEV - 1, N_DEV)
        right = lax.rem(my + 1, N_DEV)

        dirs = [
            dict(d=1, comm=comm_r, stage=stage_r, ss=ss_r, rs=rs_r,
                 ld=ld_r, st=st_r, cred=cred_r, to=right, fr=left,
                 col=0),
            dict(d=-1, comm=comm_l, stage=stage_l, ss=ss_l, rs=rs_l,
                 ld=ld_l, st=st_l, cred=cred_l, to=left, fr=right,
                 col=nh),
        ]

        def add_idx(c, s):
            return lax.rem(my - c["d"] * (s + 1) + 2 * N_DEV, N_DEV)

        def out_idx(c, s):
            if s == N_DEV - 2:
                return lax.rem(my + c["d"] + N_DEV, N_DEV)
            t = s - (N_DEV - 1)
            return lax.rem(my - c["d"] * t + 2 * N_DEV, N_DEV)

        def mk_rdma(c, s, k):
            if s == 0:
                src = x_hbm.at[pl.ds(my * ch + k * ck, ck),
                               pl.ds(c["col"], nh)]
            else:
                src = c["comm"].at[s % 2, k]
            if s == N_HOP - 1:
                row0 = lax.rem(my - c["d"] + N_DEV, N_DEV) * ch + k * ck
                dst = out_hbm.at[pl.ds(row0, ck), pl.ds(c["col"], nh)]
            else:
                dst = c["comm"].at[(s + 1) % 2, k]
            return pltpu.make_async_remote_copy(
                src_ref=src, dst_ref=dst,
                send_sem=c["ss"].at[k],
                recv_sem=c["rs"].at[k],
                device_id=(c["to"],),
                device_id_type=pl.DeviceIdType.MESH,
            )

        def mk_load(c, s, k):
            row0 = add_idx(c, s) * ch + k * ck
            return pltpu.make_async_copy(
                x_hbm.at[pl.ds(row0, ck), pl.ds(c["col"], nh)],
                c["stage"].at[k], c["ld"].at[k])

        for c in dirs:
            c["pend_ld"] = [mk_load(c, 0, k) for k in range(K)]
            for cp in c["pend_ld"]:
                cp.start()
            c["pend_st"] = [None] * K
            c["rd"] = {}

        barrier = pltpu.get_barrier_semaphore()
        for nbr in (left, right):
            pl.semaphore_signal(barrier, inc=1, device_id=(nbr,),
                                device_id_type=pl.DeviceIdType.MESH)
        for c in dirs:
            pl.semaphore_signal(c["cred"], inc=K, device_id=(c["fr"],),
                                device_id_type=pl.DeviceIdType.MESH)
        pl.semaphore_wait(barrier, 2)

        for k in range(K):
            for c in dirs:
                pl.semaphore_wait(c["cred"], 1)
                c["rd"][(0, k)] = mk_rdma(c, 0, k)
                c["rd"][(0, k)].start()

        for s in range(N_HOP):
            recv_slot = (s + 1) % 2
            for k in range(K):
                for c in dirs:
                    rd = c["rd"].pop((s, k))
                    rd.wait_send()
                    if c["pend_st"][k] is not None:
                        c["pend_st"][k].wait()
                        c["pend_st"][k] = None
                    rd.wait_recv()
                    if s < N_HOP - 1:
                        pl.semaphore_signal(
                            c["cred"], inc=1, device_id=(c["fr"],),
                            device_id_type=pl.DeviceIdType.MESH)
                    if s < N_DEV - 1:
                        c["pend_ld"][k].wait()
                        c["comm"][recv_slot, k] = (
                            c["comm"][recv_slot, k] + c["stage"][k])
                    if s < N_HOP - 1:
                        pl.semaphore_wait(c["cred"], 1)
                        c["rd"][(s + 1, k)] = mk_rdma(c, s + 1, k)
                        c["rd"][(s + 1, k)].start()
                    if s < N_DEV - 2:
                        c["pend_ld"][k] = mk_load(c, s + 1, k)
                        c["pend_ld"][k].start()
                    if N_DEV - 2 <= s < N_HOP - 1:
                        row0 = out_idx(c, s) * ch + k * ck
                        cpo = pltpu.make_async_copy(
                            c["comm"].at[recv_slot, k],
                            out_hbm.at[pl.ds(row0, ck),
                                       pl.ds(c["col"], nh)],
                            c["st"].at[k])
                        cpo.start()
                        c["pend_st"][k] = cpo

    return pl.pallas_call(
        body,
        out_shape=jax.ShapeDtypeStruct((m_per, n), x.dtype),
        in_specs=[pl.BlockSpec(memory_space=pltpu.MemorySpace.HBM)],
        out_specs=pl.BlockSpec(memory_space=pltpu.MemorySpace.HBM),
        scratch_shapes=[
            pltpu.VMEM((2, K, ck, nh), x.dtype),
            pltpu.VMEM((2, K, ck, nh), x.dtype),
            pltpu.VMEM((K, ck, nh), x.dtype),
            pltpu.VMEM((K, ck, nh), x.dtype),
            pltpu.SemaphoreType.DMA((K,)),
            pltpu.SemaphoreType.DMA((K,)),
            pltpu.SemaphoreType.DMA((K,)),
            pltpu.SemaphoreType.DMA((K,)),
            pltpu.SemaphoreType.DMA((K,)),
            pltpu.SemaphoreType.DMA((K,)),
            pltpu.SemaphoreType.DMA((K,)),
            pltpu.SemaphoreType.DMA((K,)),
            pltpu.SemaphoreType.REGULAR,
            pltpu.SemaphoreType.REGULAR,
        ],
        compiler_params=pltpu.CompilerParams(
            collective_id=0,
            vmem_limit_bytes=100 * 1024 * 1024,
        ),
    )(x)


# device time: 589763 ns/iter; 1.9835x vs baseline; 1.0000x over previous
import jax
import jax.numpy as jnp
from jax import lax
from jax.experimental import pallas as pl
from jax.experimental.pallas import tpu as pltpu

N_DEV = 4
SLICE_ROWS = (256, 768, 1280, 1792)
K = len(SLICE_ROWS)
N_HOP = 2 * (N_DEV - 1)


def kernel(x):
    m_per, n = x.shape
    ch = m_per // N_DEV
    nh = n // 2
    assert sum(SLICE_ROWS) == ch
    off = [sum(SLICE_ROWS[:k]) for k in range(K)]

    def body(x_hbm, out_hbm, *refs):
        (comm_r0, comm_r1, comm_r2, comm_r3,
         comm_l0, comm_l1, comm_l2, comm_l3,
         stage_r0, stage_r1, stage_r2, stage_r3,
         stage_l0, stage_l1, stage_l2, stage_l3,
         ss_r, rs_r, ld_r, st_r, ss_l, rs_l, ld_l, st_l,
         cred_r, cred_l) = refs
        my = lax.axis_index("i")
        left = lax.rem(my + N_DEV - 1, N_DEV)
        right = lax.rem(my + 1, N_DEV)

        dirs = [
            dict(d=1, comm=[comm_r0, comm_r1, comm_r2, comm_r3],
                 stage=[stage_r0, stage_r1, stage_r2, stage_r3],
                 ss=ss_r, rs=rs_r, ld=ld_r, st=st_r, cred=cred_r,
                 to=right, fr=left, col=0),
            dict(d=-1, comm=[comm_l0, comm_l1, comm_l2, comm_l3],
                 stage=[stage_l0, stage_l1, stage_l2, stage_l3],
                 ss=ss_l, rs=rs_l, ld=ld_l, st=st_l, cred=cred_l,
                 to=left, fr=right, col=nh),
        ]

        def add_idx(c, s):
            return lax.rem(my - c["d"] * (s + 1) + 2 * N_DEV, N_DEV)

        def out_idx(c, s):
            if s == N_DEV - 2:
                return lax.rem(my + c["d"] + N_DEV, N_DEV)
            t = s - (N_DEV - 1)
            return lax.rem(my - c["d"] * t + 2 * N_DEV, N_DEV)

        def mk_rdma(c, s, k):
            if s == 0:
                src = x_hbm.at[pl.ds(my * ch + off[k], SLICE_ROWS[k]),
                               pl.ds(c["col"], nh)]
            else:
                src = c["comm"][k].at[s % 2]
            if s == N_HOP - 1:
                row0 = (lax.rem(my - c["d"] + N_DEV, N_DEV) * ch
                        + off[k])
                dst = out_hbm.at[pl.ds(row0, SLICE_ROWS[k]),
                                 pl.ds(c["col"], nh)]
            else:
                dst = c["comm"][k].at[(s + 1) % 2]
            return pltpu.make_async_remote_copy(
                src_ref=src, dst_ref=dst,
                send_sem=c["ss"].at[k],
                recv_sem=c["rs"].at[k],
                device_id=(c["to"],),
                device_id_type=pl.DeviceIdType.MESH,
            )

        def mk_load(c, s, k):
            row0 = add_idx(c, s) * ch + off[k]
            return pltpu.make_async_copy(
                x_hbm.at[pl.ds(row0, SLICE_ROWS[k]), pl.ds(c["col"], nh)],
                c["stage"][k], c["ld"].at[k])

        for c in dirs:
            c["pend_ld"] = [mk_load(c, 0, k) for k in range(K)]
            for cp in c["pend_ld"]:
                cp.start()
            c["pend_st"] = [None] * K
            c["rd"] = {}

        barrier = pltpu.get_barrier_semaphore()
        for nbr in (left, right):
            pl.semaphore_signal(barrier, inc=1, device_id=(nbr,),
                                device_id_type=pl.DeviceIdType.MESH)
        for c in dirs:
            pl.semaphore_signal(c["cred"], inc=K, device_id=(c["fr"],),
                                device_id_type=pl.DeviceIdType.MESH)
        pl.semaphore_wait(barrier, 2)

        for k in range(K):
            for c in dirs:
                pl.semaphore_wait(c["cred"], 1)
                c["rd"][(0, k)] = mk_rdma(c, 0, k)
                c["rd"][(0, k)].start()

        for s in range(N_HOP):
            recv_slot = (s + 1) % 2
            for k in range(K):
                for c in dirs:
                    rd = c["rd"].pop((s, k))
                    rd.wait_send()
                    if c["pend_st"][k] is not None:
                        c["pend_st"][k].wait()
                        c["pend_st"][k] = None
                    rd.wait_recv()
                    if s < N_HOP - 1:
                        pl.semaphore_signal(
                            c["cred"], inc=1, device_id=(c["fr"],),
                            device_id_type=pl.DeviceIdType.MESH)
                    if s < N_DEV - 1:
                        c["pend_ld"][k].wait()
                        c["comm"][k][recv_slot] = (
                            c["comm"][k][recv_slot] + c["stage"][k][...])
                    if s < N_HOP - 1:
                        pl.semaphore_wait(c["cred"], 1)
                        c["rd"][(s + 1, k)] = mk_rdma(c, s + 1, k)
                        c["rd"][(s + 1, k)].start()
                    if s < N_DEV - 2:
                        c["pend_ld"][k] = mk_load(c, s + 1, k)
                        c["pend_ld"][k].start()
                    if N_DEV - 2 <= s < N_HOP - 1:
                        row0 = out_idx(c, s) * ch + off[k]
                        cpo = pltpu.make_async_copy(
                            c["comm"][k].at[recv_slot],
                            out_hbm.at[pl.ds(row0, SLICE_ROWS[k]),
                                       pl.ds(c["col"], nh)],
                            c["st"].at[k])
                        cpo.start()
                        c["pend_st"][k] = cpo

    comm_shapes = [pltpu.VMEM((2, r, nh), x.dtype) for r in SLICE_ROWS]
    stage_shapes = [pltpu.VMEM((r, nh), x.dtype) for r in SLICE_ROWS]
    return pl.pallas_call(
        body,
        out_shape=jax.ShapeDtypeStruct((m_per, n), x.dtype),
        in_specs=[pl.BlockSpec(memory_space=pltpu.MemorySpace.HBM)],
        out_specs=pl.BlockSpec(memory_space=pltpu.MemorySpace.HBM),
        scratch_shapes=(
            comm_shapes + comm_shapes
            + stage_shapes + stage_shapes
            + [
                pltpu.SemaphoreType.DMA((K,)),
                pltpu.SemaphoreType.DMA((K,)),
                pltpu.SemaphoreType.DMA((K,)),
                pltpu.SemaphoreType.DMA((K,)),
                pltpu.SemaphoreType.DMA((K,)),
                pltpu.SemaphoreType.DMA((K,)),
                pltpu.SemaphoreType.DMA((K,)),
                pltpu.SemaphoreType.DMA((K,)),
                pltpu.SemaphoreType.REGULAR,
                pltpu.SemaphoreType.REGULAR,
            ]
        ),
        compiler_params=pltpu.CompilerParams(
            collective_id=0,
            vmem_limit_bytes=100 * 1024 * 1024,
        ),
    )(x)


# device time: 589725 ns/iter; 1.9836x vs baseline; 1.0001x over previous
import jax
import jax.numpy as jnp
from jax import lax
from jax.experimental import pallas as pl
from jax.experimental.pallas import tpu as pltpu

N_DEV = 4
K = 4
N_HOP = 2 * (N_DEV - 1)


def kernel(x):
    m_per, n = x.shape
    ch = m_per // N_DEV
    ck = ch // K
    nh = n // 2

    def body(x_hbm, out_hbm, comm_r, comm_l, stage_r, stage_l,
             ss_r, rs_r, ld_r, st_r, ss_l, rs_l, ld_l, st_l,
             cred_r, cred_l):
        my = lax.axis_index("i")
        left = lax.rem(my + N_DEV - 1, N_DEV)
        right = lax.rem(my + 1, N_DEV)

        dirs = [
            dict(d=1, comm=comm_r, stage=stage_r, ss=ss_r, rs=rs_r,
                 ld=ld_r, st=st_r, cred=cred_r, to=right, fr=left,
                 col=0),
            dict(d=-1, comm=comm_l, stage=stage_l, ss=ss_l, rs=rs_l,
                 ld=ld_l, st=st_l, cred=cred_l, to=left, fr=right,
                 col=nh),
        ]

        def add_idx(c, s):
            return lax.rem(my - c["d"] * (s + 1) + 2 * N_DEV, N_DEV)

        def out_idx(c, s):
            if s == N_DEV - 2:
                return lax.rem(my + c["d"] + N_DEV, N_DEV)
            t = s - (N_DEV - 1)
            return lax.rem(my - c["d"] * t + 2 * N_DEV, N_DEV)

        def mk_rdma(c, s, k):
            if s == 0:
                src = x_hbm.at[pl.ds(my * ch + k * ck, ck),
                               pl.ds(c["col"], nh)]
            else:
                src = c["comm"].at[s % 2, k]
            if s == N_HOP - 1:
                row0 = lax.rem(my - c["d"] + N_DEV, N_DEV) * ch + k * ck
                dst = out_hbm.at[pl.ds(row0, ck), pl.ds(c["col"], nh)]
            else:
                dst = c["comm"].at[(s + 1) % 2, k]
            return pltpu.make_async_remote_copy(
                src_ref=src, dst_ref=dst,
                send_sem=c["ss"].at[k],
                recv_sem=c["rs"].at[k],
                device_id=(c["to"],),
                device_id_type=pl.DeviceIdType.MESH,
            )

        def mk_load(c, s, k):
            row0 = add_idx(c, s) * ch + k * ck
            return pltpu.make_async_copy(
                x_hbm.at[pl.ds(row0, ck), pl.ds(c["col"], nh)],
                c["stage"].at[k], c["ld"].at[k])

        for c in dirs:
            c["pend_ld"] = [mk_load(c, 0, k) for k in range(K)]
            for cp in c["pend_ld"]:
                cp.start()
            c["pend_st"] = [None] * K
            c["rd"] = {}

        barrier = pltpu.get_barrier_semaphore()
        for nbr in (left, right):
            pl.semaphore_signal(barrier, inc=1, device_id=(nbr,),
                                device_id_type=pl.DeviceIdType.MESH)
        for c in dirs:
            pl.semaphore_signal(c["cred"], inc=K, device_id=(c["fr"],),
                                device_id_type=pl.DeviceIdType.MESH)
        pl.semaphore_wait(barrier, 2)

        for k in range(K):
            for c in dirs:
                pl.semaphore_wait(c["cred"], 1)
                c["rd"][(0, k)] = mk_rdma(c, 0, k)
                c["rd"][(0, k)].start()

        for s in range(N_HOP):
            recv_slot = (s + 1) % 2
            for k in range(K):
                for c in dirs:
                    rd = c["rd"].pop((s, k))
                    rd.wait_send()
                    if c["pend_st"][k] is not None:
                        c["pend_st"][k].wait()
                        c["pend_st"][k] = None
                    rd.wait_recv()
                    if s < N_HOP - 1:
                        pl.semaphore_signal(
                            c["cred"], inc=1, device_id=(c["fr"],),
                            device_id_type=pl.DeviceIdType.MESH)
                    if s < N_DEV - 1:
                        c["pend_ld"][k].wait()
                        c["comm"][recv_slot, k] = (
                            c["comm"][recv_slot, k] + c["stage"][k])
                    if s < N_HOP - 1:
                        pl.semaphore_wait(c["cred"], 1)
                        c["rd"][(s + 1, k)] = mk_rdma(c, s + 1, k)
                        c["rd"][(s + 1, k)].start()
                    if s < N_DEV - 2:
                        c["pend_ld"][k] = mk_load(c, s + 1, k)
                        c["pend_ld"][k].start()
                    if N_DEV - 2 <= s < N_HOP - 1:
                        row0 = out_idx(c, s) * ch + k * ck
                        cpo = pltpu.make_async_copy(
                            c["comm"].at[recv_slot, k],
                            out_hbm.at[pl.ds(row0, ck),
                                       pl.ds(c["col"], nh)],
                            c["st"].at[k])
                        cpo.start()
                        c["pend_st"][k] = cpo

    return pl.pallas_call(
        body,
        out_shape=jax.ShapeDtypeStruct((m_per, n), x.dtype),
        in_specs=[pl.BlockSpec(memory_space=pltpu.MemorySpace.HBM)],
        out_specs=pl.BlockSpec(memory_space=pltpu.MemorySpace.HBM),
        scratch_shapes=[
            pltpu.VMEM((2, K, ck, nh), x.dtype),
            pltpu.VMEM((2, K, ck, nh), x.dtype),
            pltpu.VMEM((K, ck, nh), x.dtype),
            pltpu.VMEM((K, ck, nh), x.dtype),
            pltpu.SemaphoreType.DMA((K,)),
            pltpu.SemaphoreType.DMA((K,)),
            pltpu.SemaphoreType.DMA((K,)),
            pltpu.SemaphoreType.DMA((K,)),
            pltpu.SemaphoreType.DMA((K,)),
            pltpu.SemaphoreType.DMA((K,)),
            pltpu.SemaphoreType.DMA((K,)),
            pltpu.SemaphoreType.DMA((K,)),
            pltpu.SemaphoreType.REGULAR,
            pltpu.SemaphoreType.REGULAR,
        ],
        compiler_params=pltpu.CompilerParams(
            collective_id=0,
            vmem_limit_bytes=100 * 1024 * 1024,
        ),
    )(x)
